# baseline (speedup 1.0000x reference)
import functools

import jax
import jax.numpy as jnp
from jax import lax
from jax.experimental import pallas as pl
from jax.experimental.pallas import tpu as pltpu
from jax.experimental.pallas import tpu_sc as plsc

N_NODES = 10080
N_EDGES = 322560
BATCH = 360
NPG = 28
SEQ = 128
BL = BATCH * SEQ

NC = 2
NS = 16
NW = NC * NS
EPT = N_EDGES // NW
K_EDGE = 112
NCHUNK = EPT // K_EDGE
NPT = N_NODES // NS
NPAD = 10240
LANES = 16
EPS = 1e-5

_SC_MESH = plsc.VectorSubcoreMesh(core_axis_name="c", subcore_axis_name="s",
                                  num_cores=NC, num_subcores=NS)


@functools.partial(
    pl.kernel,
    out_type=jax.ShapeDtypeStruct((NW, NPAD), jnp.float32),
    mesh=_SC_MESH,
    scratch_types=[
        pltpu.VMEM((EPT,), jnp.int32),
        pltpu.VMEM((NPAD,), jnp.float32),
    ],
    compiler_params=pltpu.CompilerParams(needs_layout_passes=False),
)
def _deg_counts(dst_hbm, out_hbm, idx_v, hist_v):
    cid = lax.axis_index("c")
    sid = lax.axis_index("s")
    wid = cid * NS + sid

    zero16 = jnp.zeros((LANES,), jnp.float32)

    def _zero(i, c):
        hist_v[pl.ds(i * LANES, LANES)] = zero16
        return c

    lax.fori_loop(0, NPAD // LANES, _zero, 0)

    pltpu.sync_copy(dst_hbm.at[pl.ds(wid * EPT, EPT)], idx_v)

    ones16 = jnp.ones((LANES,), jnp.float32)

    def _accum(i, c):
        idx = idx_v[pl.ds(i * LANES, LANES)]
        plsc.addupdate_scatter(hist_v, [idx], ones16)
        return c

    lax.fori_loop(0, EPT // LANES, _accum, 0)

    pltpu.sync_copy(hist_v, out_hbm.at[wid])


def _make_msg_kernel(d):
    @functools.partial(
        pl.kernel,
        out_type=jax.ShapeDtypeStruct((NC, N_NODES, d), jnp.float32),
        mesh=_SC_MESH,
        scratch_types=[
            pltpu.VMEM((EPT,), jnp.int32),
            pltpu.VMEM((NCHUNK, K_EDGE), jnp.int32),
            pltpu.VMEM((K_EDGE, d), jnp.float32),
            pltpu.VMEM((K_EDGE, d), jnp.float32),
            pltpu.VMEM_SHARED((N_NODES, d), jnp.float32),
            pltpu.SemaphoreType.DMA,
            pltpu.SemaphoreType.DMA,
            pltpu.SemaphoreType.DMA,
            pltpu.SemaphoreType.DMA,
        ],
        compiler_params=pltpu.CompilerParams(needs_layout_passes=False,
                                             use_tc_tiling_on_sc=False),
    )
    def _msg(src_hbm, dst2_hbm, g_hbm, out_hbm, si_v, di_v, rows0_v, rows1_v,
             acc_sh, sem0, sem1, ssem0, ssem1):
        cid = lax.axis_index("c")
        sid = lax.axis_index("s")
        wid = cid * NS + sid
        row0 = sid * NPT

        pltpu.sync_copy(src_hbm.at[pl.ds(wid * EPT, EPT)], si_v)
        pltpu.sync_copy(dst2_hbm.at[pl.ds(wid * NCHUNK, NCHUNK)], di_v)
        pltpu.sync_copy(g_hbm.at[pl.ds(row0, NPT)], acc_sh.at[pl.ds(row0, NPT)])
        plsc.subcore_barrier()

        def _gather(j, rref, sem):
            return pltpu.make_async_copy(
                g_hbm.at[si_v.at[pl.ds(j * K_EDGE, K_EDGE)]], rref, sem)

        def _scatter(j, rref, ssem):
            return pltpu.make_async_copy(rref, acc_sh.at[di_v.at[j]], ssem)

        _gather(0, rows0_v, sem0).start()
        _gather(1, rows1_v, sem1).start()

        def _step(j, rref, sem, ssem):
            _gather(j, rref, sem).wait()
            _scatter(j, rref, ssem).start(add=True)

            @pl.when(j + 2 < NCHUNK)
            def _():
                _scatter(j, rref, ssem).wait()
                _gather(j + 2, rref, sem).start()

        def _chunk2(j2, c):
            _step(j2 * 2, rows0_v, sem0, ssem0)
            _step(j2 * 2 + 1, rows1_v, sem1, ssem1)
            return c

        lax.fori_loop(0, NCHUNK // 2, _chunk2, 0)
        _scatter(NCHUNK - 2, rows0_v, ssem0).wait()
        _scatter(NCHUNK - 1, rows1_v, ssem1).wait()
        plsc.subcore_barrier()

        pltpu.sync_copy(acc_sh.at[pl.ds(row0, NPT)],
                        out_hbm.at[cid, pl.ds(row0, NPT)])

    return _msg


_msg128 = _make_msg_kernel(128)


def _bdot(a, b):
    return jnp.dot(a.astype(jnp.bfloat16), b.astype(jnp.bfloat16),
                   preferred_element_type=jnp.float32)


def _t0_body(x_ref, w_ref, dis_ref, out_ref):
    out_ref[...] = _bdot(x_ref[...], w_ref[...]) * dis_ref[...]


def _t0(x, w, dis):
    return pl.pallas_call(
        _t0_body,
        out_shape=jax.ShapeDtypeStruct((N_NODES, w.shape[1]), jnp.float32),
    )(x, w, dis)


def _gcn_post(p0, p1, g, dis, b, gam, bet):
    s = (p0 + p1 - g) * dis + b
    m = jnp.mean(s, axis=0, keepdims=True)
    c = s - m
    v = jnp.mean(c * c, axis=0, keepdims=True)
    return jnp.maximum(gam * c * lax.rsqrt(v + EPS) + bet, 0.0)


def _tmid_body(p0_ref, p1_ref, g_ref, dis_ref, b_ref, gam_ref, bet_ref,
               w_ref, out_ref):
    h = _gcn_post(p0_ref[...], p1_ref[...], g_ref[...], dis_ref[...],
                  b_ref[...], gam_ref[...], bet_ref[...])
    out_ref[...] = _bdot(h, w_ref[...]) * dis_ref[...]


def _tmid(p0, p1, g, dis, b, gam, bet, w):
    return pl.pallas_call(
        _tmid_body,
        out_shape=jax.ShapeDtypeStruct((N_NODES, w.shape[1]), jnp.float32),
    )(p0, p1, g, dis, b, gam, bet, w)


def _tlast_body(p0_ref, p1_ref, g_ref, dis_ref, b_ref, gam_ref, bet_ref,
                out_ref):
    out_ref[...] = _gcn_post(p0_ref[...], p1_ref[...], g_ref[...], dis_ref[...],
                             b_ref[...], gam_ref[...], bet_ref[...])


def _tlast(p0, p1, g, dis, b, gam, bet):
    return pl.pallas_call(
        _tlast_body,
        out_shape=jax.ShapeDtypeStruct((N_NODES, 128), jnp.float32),
    )(p0, p1, g, dis, b, gam, bet)


CB = 5760
NCB = BL // CB


def _shift_cols(x, s):
    if s == 0:
        return x
    w = x.shape[1]
    z = jnp.zeros((x.shape[0], s), jnp.float32)
    xs = jnp.concatenate([z, x[:, :w - s]], axis=1)
    col = lax.broadcasted_iota(jnp.int32, (1, w), 1)
    return xs * (col % SEQ >= s).astype(jnp.float32)


def _conv_cf(x, wk, b, dil):
    o = b
    for k in range(3):
        o = o + jnp.dot(wk[k], _shift_cols(x, (2 - k) * dil),
                        preferred_element_type=jnp.float32)
    return o


def _bn_apply(x, s1, s2, gam, bet):
    m = s1 * (1.0 / BL)
    v = s2 * (1.0 / BL) - m * m
    return jnp.maximum(gam * (x - m) * lax.rsqrt(v + EPS) + bet, 0.0)


def _make_conv_stats(cin, cout, dil, pre_bn):
    def _body(*refs):
        if pre_bn:
            (x_ref, ps1_ref, ps2_ref, pg_ref, pb_ref,
             w0_ref, w1_ref, w2_ref, b_ref, o_ref, s1_ref, s2_ref) = refs
            x = _bn_apply(x_ref[...], ps1_ref[...], ps2_ref[...],
                          pg_ref[...], pb_ref[...])
        else:
            x_ref, w0_ref, w1_ref, w2_ref, b_ref, o_ref, s1_ref, s2_ref = refs
            x = x_ref[...]
        o = _conv_cf(x, [w0_ref[...], w1_ref[...], w2_ref[...]],
                     b_ref[...], dil)
        o_ref[...] = o

        @pl.when(pl.program_id(0) == 0)
        def _():
            s1_ref[...] = jnp.zeros_like(s1_ref)
            s2_ref[...] = jnp.zeros_like(s2_ref)

        s1_ref[...] += jnp.sum(o, axis=1, keepdims=True)
        s2_ref[...] += jnp.sum(o * o, axis=1, keepdims=True)

    col_spec = lambda c: pl.BlockSpec((c, CB), lambda j: (0, j))
    full = lambda a, b: pl.BlockSpec((a, b), lambda j: (0, 0))
    in_specs = [col_spec(cin)]
    if pre_bn:
        in_specs += [full(cin, 1)] * 4
    in_specs += [full(cout, cin)] * 3 + [full(cout, 1)]

    def _call(x, w3, b, pre=()):
        return pl.pallas_call(
            _body,
            grid=(NCB,),
            in_specs=in_specs,
            out_specs=[col_spec(cout), full(cout, 1), full(cout, 1)],
            out_shape=[jax.ShapeDtypeStruct((cout, BL), jnp.float32),
                       jax.ShapeDtypeStruct((cout, 1), jnp.float32),
                       jax.ShapeDtypeStruct((cout, 1), jnp.float32)],
        )(x, *pre, w3[:, :, 0], w3[:, :, 1], w3[:, :, 2], b[:, None])

    return _call


def _make_res_combine(cin, cout):
    def _body(o2_ref, s1_ref, s2_ref, g_ref, be_ref, x_ref, wd_ref, bd_ref,
              out_ref):
        a2 = _bn_apply(o2_ref[...], s1_ref[...], s2_ref[...],
                       g_ref[...], be_ref[...])
        res = jnp.dot(wd_ref[...], x_ref[...],
                      preferred_element_type=jnp.float32) + bd_ref[...]
        out_ref[...] = jnp.maximum(a2 + res, 0.0)

    col_spec = lambda c: pl.BlockSpec((c, CB), lambda j: (0, j))
    full = lambda a, b: pl.BlockSpec((a, b), lambda j: (0, 0))

    def _call(o2, s1, s2, g, be, x, wd, bd):
        return pl.pallas_call(
            _body,
            grid=(NCB,),
            in_specs=[col_spec(cout), full(cout, 1), full(cout, 1),
                      full(cout, 1), full(cout, 1), col_spec(cin),
                      full(cout, cin), full(cout, 1)],
            out_specs=col_spec(cout),
            out_shape=jax.ShapeDtypeStruct((cout, BL), jnp.float32),
        )(o2, s1, s2, g, be, x, wd, bd)

    return _call


def _make_tblock(cin, cout, dil):
    conv1 = _make_conv_stats(cin, cout, dil, pre_bn=False)
    conv2 = _make_conv_stats(cout, cout, dil, pre_bn=True)
    comb = _make_res_combine(cin, cout)

    def _call(x, p, pre):
        o1, a1, a2 = conv1(x, p[pre + '_w1'], p[pre + '_b1'])
        o2, c1, c2 = conv2(o1, p[pre + '_w2'], p[pre + '_b2'],
                           pre=(a1, a2, p[pre + '_bn1_g'][:, None],
                                p[pre + '_bn1_b'][:, None]))
        return comb(o2, c1, c2, p[pre + '_bn2_g'][:, None],
                    p[pre + '_bn2_b'][:, None], x,
                    p[pre + '_down_w'][:, :, 0], p[pre + '_down_b'][:, None])

    return _call


_tblock0 = _make_tblock(NPG, 128, 1)
_tblock1 = _make_tblock(128, 64, 2)
_tblock2 = _make_tblock(64, 32, 4)


def _head_body(t_ref, w1_ref, b1_ref, w2_ref, b2_ref, out_ref):
    wc = jnp.dot(w1_ref[...], w2_ref[...], preferred_element_type=jnp.float32)
    bc = jnp.dot(b1_ref[...], w2_ref[...],
                 preferred_element_type=jnp.float32) + b2_ref[...]
    out_ref[...] = jnp.dot(t_ref[...], wc,
                           preferred_element_type=jnp.float32) + bc


def _head(t, w1, b1, w2, b2):
    return pl.pallas_call(
        _head_body,
        out_shape=jax.ShapeDtypeStruct((BATCH, 1), jnp.float32),
    )(t, w1, b1, w2, b2)


def kernel(x, edge_index, params):
    p = params
    src = edge_index[0]
    dst = edge_index[1]
    dst2 = dst.reshape(N_EDGES // K_EDGE, K_EDGE)

    deg_part = _deg_counts(dst)
    deg = jnp.sum(deg_part[:, :N_NODES], axis=0) + 1.0
    dis = lax.rsqrt(deg)[:, None]

    pad64 = lambda a: jnp.pad(a, ((0, 0), (0, 64)))
    g1 = _t0(x, pad64(p['gcn1_w']), dis)
    pt = _msg128(src, dst2, g1)
    g2 = _tmid(pt[0], pt[1], g1, dis, pad64(p['gcn1_b'][None, :]),
               pad64(p['bn1_g'][None, :]), pad64(p['bn1_b'][None, :]),
               jnp.pad(p['gcn2_w'], ((0, 64), (0, 0))))
    pt = _msg128(src, dst2, g2)
    g3 = _tmid(pt[0], pt[1], g2, dis, p['gcn2_b'][None, :],
               p['bn2_g'][None, :], p['bn2_b'][None, :], p['gcn3_w'])
    pt = _msg128(src, dst2, g3)
    h3 = _tlast(pt[0], pt[1], g3, dis, p['gcn3_b'][None, :],
                p['bn3_g'][None, :], p['bn3_b'][None, :])

    x0 = h3.reshape(BATCH, NPG, SEQ).transpose(1, 0, 2).reshape(NPG, BL)
    x1 = _tblock0(x0, p, 'tcn0')
    x2 = _tblock1(x1, p, 'tcn1')
    x3 = _tblock2(x2, p, 'tcn2')

    t = x3.reshape(32, BATCH, SEQ)[:, :, SEQ - 1].T
    return _head(t, p['fc1_w'], p['fc1_b'][None, :], p['fc_w'],
                 p['fc_b'][None, :])


# speedup vs baseline: 1.0967x; 1.0967x over previous
import functools

import jax
import jax.numpy as jnp
from jax import lax
from jax.experimental import pallas as pl
from jax.experimental.pallas import tpu as pltpu
from jax.experimental.pallas import tpu_sc as plsc

N_NODES = 10080
N_EDGES = 322560
BATCH = 360
NPG = 28
SEQ = 128
BL = BATCH * SEQ

NC = 2
NS = 16
NW = NC * NS
EPT = N_EDGES // NW
K_EDGE = 112
NCHUNK = EPT // K_EDGE
NPT = N_NODES // NS
NPAD = 10240
LANES = 16
EPS = 1e-5

_SC_MESH = plsc.VectorSubcoreMesh(core_axis_name="c", subcore_axis_name="s",
                                  num_cores=NC, num_subcores=NS)


@functools.partial(
    pl.kernel,
    out_type=jax.ShapeDtypeStruct((NW, NPAD), jnp.float32),
    mesh=_SC_MESH,
    scratch_types=[
        pltpu.VMEM((EPT,), jnp.int32),
        pltpu.VMEM((NPAD,), jnp.float32),
    ],
    compiler_params=pltpu.CompilerParams(needs_layout_passes=False),
)
def _deg_counts(dst_hbm, out_hbm, idx_v, hist_v):
    cid = lax.axis_index("c")
    sid = lax.axis_index("s")
    wid = cid * NS + sid

    zero16 = jnp.zeros((LANES,), jnp.float32)

    def _zero(i, c):
        hist_v[pl.ds(i * LANES, LANES)] = zero16
        return c

    lax.fori_loop(0, NPAD // LANES, _zero, 0)

    pltpu.sync_copy(dst_hbm.at[pl.ds(wid * EPT, EPT)], idx_v)

    ones16 = jnp.ones((LANES,), jnp.float32)

    def _accum(i, c):
        idx = idx_v[pl.ds(i * LANES, LANES)]
        plsc.addupdate_scatter(hist_v, [idx], ones16)
        return c

    lax.fori_loop(0, EPT // LANES, _accum, 0)

    pltpu.sync_copy(hist_v, out_hbm.at[wid])


def _make_msg_kernel(d):
    @functools.partial(
        pl.kernel,
        out_type=jax.ShapeDtypeStruct((NC, N_NODES, d), jnp.float32),
        mesh=_SC_MESH,
        scratch_types=[
            pltpu.VMEM((EPT,), jnp.int32),
            pltpu.VMEM((NCHUNK, K_EDGE), jnp.int32),
            pltpu.VMEM((K_EDGE, d), jnp.float32),
            pltpu.VMEM((K_EDGE, d), jnp.float32),
            pltpu.VMEM_SHARED((N_NODES, d), jnp.float32),
            pltpu.SemaphoreType.DMA,
            pltpu.SemaphoreType.DMA,
            pltpu.SemaphoreType.DMA,
            pltpu.SemaphoreType.DMA,
        ],
        compiler_params=pltpu.CompilerParams(needs_layout_passes=False,
                                             use_tc_tiling_on_sc=False),
    )
    def _msg(src_hbm, dst2_hbm, g_hbm, out_hbm, si_v, di_v, rows0_v, rows1_v,
             acc_sh, sem0, sem1, ssem0, ssem1):
        cid = lax.axis_index("c")
        sid = lax.axis_index("s")
        wid = cid * NS + sid
        row0 = sid * NPT

        pltpu.sync_copy(src_hbm.at[pl.ds(wid * EPT, EPT)], si_v)
        pltpu.sync_copy(dst2_hbm.at[pl.ds(wid * NCHUNK, NCHUNK)], di_v)
        pltpu.sync_copy(g_hbm.at[pl.ds(row0, NPT)], acc_sh.at[pl.ds(row0, NPT)])
        plsc.subcore_barrier()

        def _gather(j, rref, sem):
            return pltpu.make_async_copy(
                g_hbm.at[si_v.at[pl.ds(j * K_EDGE, K_EDGE)]], rref, sem)

        def _scatter(j, rref, ssem):
            return pltpu.make_async_copy(rref, acc_sh.at[di_v.at[j]], ssem)

        _gather(0, rows0_v, sem0).start()
        _gather(1, rows1_v, sem1).start()

        def _step(j, rref, sem, ssem):
            _gather(j, rref, sem).wait()
            _scatter(j, rref, ssem).start(add=True)

            @pl.when(j + 2 < NCHUNK)
            def _():
                _scatter(j, rref, ssem).wait()
                _gather(j + 2, rref, sem).start()

        def _chunk2(j2, c):
            _step(j2 * 2, rows0_v, sem0, ssem0)
            _step(j2 * 2 + 1, rows1_v, sem1, ssem1)
            return c

        lax.fori_loop(0, NCHUNK // 2, _chunk2, 0)
        _scatter(NCHUNK - 2, rows0_v, ssem0).wait()
        _scatter(NCHUNK - 1, rows1_v, ssem1).wait()
        plsc.subcore_barrier()

        pltpu.sync_copy(acc_sh.at[pl.ds(row0, NPT)],
                        out_hbm.at[cid, pl.ds(row0, NPT)])

    return _msg


_msg128 = _make_msg_kernel(128)


def _bdot(a, b):
    return jnp.dot(a.astype(jnp.bfloat16), b.astype(jnp.bfloat16),
                   preferred_element_type=jnp.float32)


def _t0_body(x_ref, w_ref, dis_ref, out_ref):
    out_ref[...] = _bdot(x_ref[...], w_ref[...]) * dis_ref[...]


def _t0(x, w, dis):
    return pl.pallas_call(
        _t0_body,
        out_shape=jax.ShapeDtypeStruct((N_NODES, w.shape[1]), jnp.float32),
    )(x, w, dis)


def _gcn_post(p0, p1, g, dis, b, gam, bet):
    s = (p0 + p1 - g) * dis + b
    m = jnp.mean(s, axis=0, keepdims=True)
    c = s - m
    v = jnp.mean(c * c, axis=0, keepdims=True)
    return jnp.maximum(gam * c * lax.rsqrt(v + EPS) + bet, 0.0)


def _tmid_body(p0_ref, p1_ref, g_ref, dis_ref, b_ref, gam_ref, bet_ref,
               w_ref, out_ref):
    h = _gcn_post(p0_ref[...], p1_ref[...], g_ref[...], dis_ref[...],
                  b_ref[...], gam_ref[...], bet_ref[...])
    out_ref[...] = _bdot(h, w_ref[...]) * dis_ref[...]


def _tmid(p0, p1, g, dis, b, gam, bet, w):
    return pl.pallas_call(
        _tmid_body,
        out_shape=jax.ShapeDtypeStruct((N_NODES, w.shape[1]), jnp.float32),
    )(p0, p1, g, dis, b, gam, bet, w)


def _tlast_body(p0_ref, p1_ref, g_ref, dis_ref, b_ref, gam_ref, bet_ref,
                out_ref):
    out_ref[...] = _gcn_post(p0_ref[...], p1_ref[...], g_ref[...], dis_ref[...],
                             b_ref[...], gam_ref[...], bet_ref[...])


def _tlast(p0, p1, g, dis, b, gam, bet):
    return pl.pallas_call(
        _tlast_body,
        out_shape=jax.ShapeDtypeStruct((N_NODES, 128), jnp.float32),
    )(p0, p1, g, dis, b, gam, bet)


CB = 9216
NCB = BL // CB


def _shift_cols(x, s):
    if s == 0:
        return x
    w = x.shape[1]
    z = jnp.zeros((x.shape[0], s), jnp.float32)
    xs = jnp.concatenate([z, x[:, :w - s]], axis=1)
    col = lax.broadcasted_iota(jnp.int32, (1, w), 1)
    return xs * (col % SEQ >= s).astype(jnp.float32)


def _conv_cf(x, wk, b, dil):
    o = b
    for k in range(3):
        o = o + jnp.dot(wk[k], _shift_cols(x, (2 - k) * dil),
                        preferred_element_type=jnp.float32)
    return o


def _bn_apply(x, s1, s2, gam, bet):
    m = s1 * (1.0 / BL)
    v = s2 * (1.0 / BL) - m * m
    return jnp.maximum(gam * (x - m) * lax.rsqrt(v + EPS) + bet, 0.0)


CBB = CB // SEQ


def _shifted_cat(x, dil):
    return jnp.concatenate(
        [_shift_cols(x, 2 * dil), _shift_cols(x, dil), x], axis=0)


def _conv_body(xin, w_ref, b_ref, dil, o_ref, s1_ref, s2_ref):
    o = jnp.dot(w_ref[...], _shifted_cat(xin, dil),
                preferred_element_type=jnp.float32) + b_ref[...]
    o_ref[...] = o

    @pl.when(pl.program_id(0) == 0)
    def _():
        s1_ref[...] = jnp.zeros_like(s1_ref)
        s2_ref[...] = jnp.zeros_like(s2_ref)

    s1_ref[...] += jnp.sum(o, axis=1, keepdims=True)
    s2_ref[...] += jnp.sum(o * o, axis=1, keepdims=True)


def _col_spec(c):
    return pl.BlockSpec((c, CB), lambda j: (0, j))


def _full(a, b):
    return pl.BlockSpec((a, b), lambda j: (0, 0))


def _comb(o2_ref, s1_ref, s2_ref, g_ref, be_ref, x_ref, wd_ref, bd_ref):
    a2 = _bn_apply(o2_ref[...], s1_ref[...], s2_ref[...],
                   g_ref[...], be_ref[...])
    res = jnp.dot(wd_ref[...], x_ref[...],
                  preferred_element_type=jnp.float32) + bd_ref[...]
    return jnp.maximum(a2 + res, 0.0)


def _make_conv_a(cin, cout, dil):
    def _body(x_ref, w_ref, b_ref, o_ref, s1_ref, s2_ref):
        _conv_body(x_ref[...], w_ref, b_ref, dil, o_ref, s1_ref, s2_ref)

    def _call(x, w, b):
        return pl.pallas_call(
            _body, grid=(NCB,),
            in_specs=[_col_spec(cin), _full(cout, 3 * cin), _full(cout, 1)],
            out_specs=[_col_spec(cout), _full(cout, 1), _full(cout, 1)],
            out_shape=[jax.ShapeDtypeStruct((cout, BL), jnp.float32),
                       jax.ShapeDtypeStruct((cout, 1), jnp.float32),
                       jax.ShapeDtypeStruct((cout, 1), jnp.float32)],
        )(x, w, b)

    return _call


def _make_conv_b(cin, dil):
    def _body(x_ref, ps1_ref, ps2_ref, pg_ref, pb_ref, w_ref, b_ref,
              o_ref, s1_ref, s2_ref):
        xin = _bn_apply(x_ref[...], ps1_ref[...], ps2_ref[...],
                        pg_ref[...], pb_ref[...])
        _conv_body(xin, w_ref, b_ref, dil, o_ref, s1_ref, s2_ref)

    def _call(o1, s1, s2, pg, pb, w, b):
        return pl.pallas_call(
            _body, grid=(NCB,),
            in_specs=[_col_spec(cin)] + [_full(cin, 1)] * 4
            + [_full(cin, 3 * cin), _full(cin, 1)],
            out_specs=[_col_spec(cin), _full(cin, 1), _full(cin, 1)],
            out_shape=[jax.ShapeDtypeStruct((cin, BL), jnp.float32),
                       jax.ShapeDtypeStruct((cin, 1), jnp.float32),
                       jax.ShapeDtypeStruct((cin, 1), jnp.float32)],
        )(o1, s1, s2, pg, pb, w, b)

    return _call


def _make_conv_c(cres, cin, cout, dil):
    def _body(o2_ref, ps1_ref, ps2_ref, pg_ref, pb_ref, xr_ref, wd_ref,
              bd_ref, w_ref, b_ref, xin_ref, o_ref, s1_ref, s2_ref):
        xin = _comb(o2_ref, ps1_ref, ps2_ref, pg_ref, pb_ref, xr_ref,
                    wd_ref, bd_ref)
        xin_ref[...] = xin
        _conv_body(xin, w_ref, b_ref, dil, o_ref, s1_ref, s2_ref)

    def _call(o2, s1, s2, pg, pb, xr, wd, bd, w, b):
        return pl.pallas_call(
            _body, grid=(NCB,),
            in_specs=[_col_spec(cin)] + [_full(cin, 1)] * 4
            + [_col_spec(cres), _full(cin, cres), _full(cin, 1),
               _full(cout, 3 * cin), _full(cout, 1)],
            out_specs=[_col_spec(cin), _col_spec(cout), _full(cout, 1),
                       _full(cout, 1)],
            out_shape=[jax.ShapeDtypeStruct((cin, BL), jnp.float32),
                       jax.ShapeDtypeStruct((cout, BL), jnp.float32),
                       jax.ShapeDtypeStruct((cout, 1), jnp.float32),
                       jax.ShapeDtypeStruct((cout, 1), jnp.float32)],
        )(o2, s1, s2, pg, pb, xr, wd, bd, w, b)

    return _call


def _tail_body(o2_ref, ps1_ref, ps2_ref, pg_ref, pb_ref, xr_ref, wd_ref,
               bd_ref, w1_ref, b1_ref, w2_ref, b2_ref, out_ref):
    xin = _comb(o2_ref, ps1_ref, ps2_ref, pg_ref, pb_ref, xr_ref,
                wd_ref, bd_ref)
    ri = lax.broadcasted_iota(jnp.int32, (CB, CBB), 0)
    bi = lax.broadcasted_iota(jnp.int32, (CB, CBB), 1)
    sel = ((ri // SEQ == bi) & (ri % SEQ == SEQ - 1)).astype(jnp.float32)
    t = jnp.dot(xin, sel, preferred_element_type=jnp.float32)
    wc = jnp.dot(w1_ref[...], w2_ref[...],
                 preferred_element_type=jnp.float32)
    bc = jnp.dot(b1_ref[...], w2_ref[...],
                 preferred_element_type=jnp.float32) + b2_ref[...]
    out_ref[...] = lax.dot_general(
        t, wc, (((0,), (0,)), ((), ())),
        preferred_element_type=jnp.float32) + bc


def _tail(o2, s1, s2, pg, pb, xr, wd, bd, w1, b1, w2, b2):
    return pl.pallas_call(
        _tail_body, grid=(NCB,),
        in_specs=[_col_spec(32)] + [_full(32, 1)] * 4
        + [_col_spec(64), _full(32, 64), _full(32, 1),
           _full(32, 128), _full(1, 128), _full(128, 1), _full(1, 1)],
        out_specs=pl.BlockSpec((CBB, 1), lambda j: (j, 0)),
        out_shape=jax.ShapeDtypeStruct((BATCH, 1), jnp.float32),
    )(o2, s1, s2, pg, pb, xr, wd, bd, w1, b1, w2, b2)


_conv_a0 = _make_conv_a(NPG, 128, 1)
_conv_b0 = _make_conv_b(128, 1)
_conv_c1 = _make_conv_c(NPG, 128, 64, 2)
_conv_b1 = _make_conv_b(64, 2)
_conv_c2 = _make_conv_c(128, 64, 32, 4)
_conv_b2 = _make_conv_b(32, 4)


def _wcat(w3):
    return jnp.concatenate([w3[:, :, 0], w3[:, :, 1], w3[:, :, 2]], axis=1)


def _tcn_head(x0, p):
    o1, a1, a2 = _conv_a0(x0, _wcat(p['tcn0_w1']), p['tcn0_b1'][:, None])
    o2, b1, b2 = _conv_b0(o1, a1, a2, p['tcn0_bn1_g'][:, None],
                          p['tcn0_bn1_b'][:, None], _wcat(p['tcn0_w2']),
                          p['tcn0_b2'][:, None])
    x1, o3, c1, c2 = _conv_c1(o2, b1, b2, p['tcn0_bn2_g'][:, None],
                              p['tcn0_bn2_b'][:, None], x0,
                              p['tcn0_down_w'][:, :, 0],
                              p['tcn0_down_b'][:, None],
                              _wcat(p['tcn1_w1']), p['tcn1_b1'][:, None])
    o4, d1, d2 = _conv_b1(o3, c1, c2, p['tcn1_bn1_g'][:, None],
                          p['tcn1_bn1_b'][:, None], _wcat(p['tcn1_w2']),
                          p['tcn1_b2'][:, None])
    x2, o5, e1, e2 = _conv_c2(o4, d1, d2, p['tcn1_bn2_g'][:, None],
                              p['tcn1_bn2_b'][:, None], x1,
                              p['tcn1_down_w'][:, :, 0],
                              p['tcn1_down_b'][:, None],
                              _wcat(p['tcn2_w1']), p['tcn2_b1'][:, None])
    o6, f1, f2 = _conv_b2(o5, e1, e2, p['tcn2_bn1_g'][:, None],
                          p['tcn2_bn1_b'][:, None], _wcat(p['tcn2_w2']),
                          p['tcn2_b2'][:, None])
    ot = _tail(o6, f1, f2, p['tcn2_bn2_g'][:, None], p['tcn2_bn2_b'][:, None],
               x2, p['tcn2_down_w'][:, :, 0], p['tcn2_down_b'][:, None],
               p['fc1_w'], p['fc1_b'][None, :], p['fc_w'], p['fc_b'][None, :])
    return ot


def kernel(x, edge_index, params):
    p = params
    src = edge_index[0]
    dst = edge_index[1]
    dst2 = dst.reshape(N_EDGES // K_EDGE, K_EDGE)

    deg_part = _deg_counts(dst)
    deg = jnp.sum(deg_part[:, :N_NODES], axis=0) + 1.0
    dis = lax.rsqrt(deg)[:, None]

    pad64 = lambda a: jnp.pad(a, ((0, 0), (0, 64)))
    g1 = _t0(x, pad64(p['gcn1_w']), dis)
    pt = _msg128(src, dst2, g1)
    g2 = _tmid(pt[0], pt[1], g1, dis, pad64(p['gcn1_b'][None, :]),
               pad64(p['bn1_g'][None, :]), pad64(p['bn1_b'][None, :]),
               jnp.pad(p['gcn2_w'], ((0, 64), (0, 0))))
    pt = _msg128(src, dst2, g2)
    g3 = _tmid(pt[0], pt[1], g2, dis, p['gcn2_b'][None, :],
               p['bn2_g'][None, :], p['bn2_b'][None, :], p['gcn3_w'])
    pt = _msg128(src, dst2, g3)
    h3 = _tlast(pt[0], pt[1], g3, dis, p['gcn3_b'][None, :],
                p['bn3_g'][None, :], p['bn3_b'][None, :])

    x0 = h3.reshape(BATCH, NPG, SEQ).transpose(1, 0, 2).reshape(NPG, BL)
    return _tcn_head(x0, p)


# speedup vs baseline: 1.0973x; 1.0005x over previous
import functools

import jax
import jax.numpy as jnp
from jax import lax
from jax.experimental import pallas as pl
from jax.experimental.pallas import tpu as pltpu
from jax.experimental.pallas import tpu_sc as plsc

N_NODES = 10080
N_EDGES = 322560
BATCH = 360
NPG = 28
SEQ = 128
BL = BATCH * SEQ

NC = 2
NS = 16
NW = NC * NS
EPT = N_EDGES // NW
K_EDGE = 112
NCHUNK = EPT // K_EDGE
NPT = N_NODES // NS
NPAD = 10240
LANES = 16
EPS = 1e-5

_SC_MESH = plsc.VectorSubcoreMesh(core_axis_name="c", subcore_axis_name="s",
                                  num_cores=NC, num_subcores=NS)


@functools.partial(
    pl.kernel,
    out_type=jax.ShapeDtypeStruct((NW, NPAD), jnp.float32),
    mesh=_SC_MESH,
    scratch_types=[
        pltpu.VMEM((EPT,), jnp.int32),
        pltpu.VMEM((NPAD,), jnp.float32),
    ],
    compiler_params=pltpu.CompilerParams(needs_layout_passes=False),
)
def _deg_counts(dst_hbm, out_hbm, idx_v, hist_v):
    cid = lax.axis_index("c")
    sid = lax.axis_index("s")
    wid = cid * NS + sid

    zero16 = jnp.zeros((LANES,), jnp.float32)

    def _zero(i, c):
        hist_v[pl.ds(i * LANES, LANES)] = zero16
        return c

    lax.fori_loop(0, NPAD // LANES, _zero, 0)

    pltpu.sync_copy(dst_hbm.at[pl.ds(wid * EPT, EPT)], idx_v)

    ones16 = jnp.ones((LANES,), jnp.float32)

    def _accum(i, c):
        idx = idx_v[pl.ds(i * LANES, LANES)]
        plsc.addupdate_scatter(hist_v, [idx], ones16)
        return c

    lax.fori_loop(0, EPT // LANES, _accum, 0)

    pltpu.sync_copy(hist_v, out_hbm.at[wid])


def _make_msg_kernel(d):
    @functools.partial(
        pl.kernel,
        out_type=jax.ShapeDtypeStruct((NC, N_NODES, d), jnp.float32),
        mesh=_SC_MESH,
        scratch_types=[
            pltpu.VMEM((EPT,), jnp.int32),
            pltpu.VMEM((NCHUNK, K_EDGE), jnp.int32),
            pltpu.VMEM((K_EDGE, d), jnp.float32),
            pltpu.VMEM((K_EDGE, d), jnp.float32),
            pltpu.VMEM_SHARED((N_NODES, d), jnp.float32),
            pltpu.SemaphoreType.DMA,
            pltpu.SemaphoreType.DMA,
            pltpu.SemaphoreType.DMA,
            pltpu.SemaphoreType.DMA,
        ],
        compiler_params=pltpu.CompilerParams(needs_layout_passes=False,
                                             use_tc_tiling_on_sc=False),
    )
    def _msg(src_hbm, dst2_hbm, g_hbm, out_hbm, si_v, di_v, rows0_v, rows1_v,
             acc_sh, sem0, sem1, ssem0, ssem1):
        cid = lax.axis_index("c")
        sid = lax.axis_index("s")
        wid = cid * NS + sid
        row0 = sid * NPT

        pltpu.sync_copy(src_hbm.at[pl.ds(wid * EPT, EPT)], si_v)
        pltpu.sync_copy(dst2_hbm.at[pl.ds(wid * NCHUNK, NCHUNK)], di_v)
        pltpu.sync_copy(g_hbm.at[pl.ds(row0, NPT)], acc_sh.at[pl.ds(row0, NPT)])
        plsc.subcore_barrier()

        def _gather(j, rref, sem):
            return pltpu.make_async_copy(
                g_hbm.at[si_v.at[pl.ds(j * K_EDGE, K_EDGE)]], rref, sem)

        def _scatter(j, rref, ssem):
            return pltpu.make_async_copy(rref, acc_sh.at[di_v.at[j]], ssem)

        _gather(0, rows0_v, sem0).start()
        _gather(1, rows1_v, sem1).start()

        def _step(j, rref, sem, ssem):
            _gather(j, rref, sem).wait()
            _scatter(j, rref, ssem).start(add=True)

            @pl.when(j + 2 < NCHUNK)
            def _():
                _scatter(j, rref, ssem).wait()
                _gather(j + 2, rref, sem).start()

        def _chunk2(j2, c):
            _step(j2 * 2, rows0_v, sem0, ssem0)
            _step(j2 * 2 + 1, rows1_v, sem1, ssem1)
            return c

        lax.fori_loop(0, NCHUNK // 2, _chunk2, 0)
        _scatter(NCHUNK - 2, rows0_v, ssem0).wait()
        _scatter(NCHUNK - 1, rows1_v, ssem1).wait()
        plsc.subcore_barrier()

        pltpu.sync_copy(acc_sh.at[pl.ds(row0, NPT)],
                        out_hbm.at[cid, pl.ds(row0, NPT)])

    return _msg


_msg128 = _make_msg_kernel(128)


def _bdot(a, b):
    return jnp.dot(a.astype(jnp.bfloat16), b.astype(jnp.bfloat16),
                   preferred_element_type=jnp.float32)


def _t0_body(x_ref, w_ref, dis_ref, out_ref):
    out_ref[...] = _bdot(x_ref[...], w_ref[...]) * dis_ref[...]


def _t0(x, w, dis):
    return pl.pallas_call(
        _t0_body,
        out_shape=jax.ShapeDtypeStruct((N_NODES, w.shape[1]), jnp.float32),
    )(x, w, dis)


def _gcn_post(p0, p1, g, dis, b, gam, bet):
    s = (p0 + p1 - g) * dis + b
    m = jnp.mean(s, axis=0, keepdims=True)
    c = s - m
    v = jnp.mean(c * c, axis=0, keepdims=True)
    return jnp.maximum(gam * c * lax.rsqrt(v + EPS) + bet, 0.0)


def _tmid_body(p0_ref, p1_ref, g_ref, dis_ref, b_ref, gam_ref, bet_ref,
               w_ref, out_ref):
    h = _gcn_post(p0_ref[...], p1_ref[...], g_ref[...], dis_ref[...],
                  b_ref[...], gam_ref[...], bet_ref[...])
    out_ref[...] = _bdot(h, w_ref[...]) * dis_ref[...]


def _tmid(p0, p1, g, dis, b, gam, bet, w):
    return pl.pallas_call(
        _tmid_body,
        out_shape=jax.ShapeDtypeStruct((N_NODES, w.shape[1]), jnp.float32),
    )(p0, p1, g, dis, b, gam, bet, w)


def _tlast_body(p0_ref, p1_ref, g_ref, dis_ref, b_ref, gam_ref, bet_ref,
                out_ref):
    out_ref[...] = _gcn_post(p0_ref[...], p1_ref[...], g_ref[...], dis_ref[...],
                             b_ref[...], gam_ref[...], bet_ref[...])


def _tlast(p0, p1, g, dis, b, gam, bet):
    return pl.pallas_call(
        _tlast_body,
        out_shape=jax.ShapeDtypeStruct((N_NODES, 128), jnp.float32),
    )(p0, p1, g, dis, b, gam, bet)


CB = 9216
NCB = BL // CB


def _shift_cols(x, s):
    if s == 0:
        return x
    w = x.shape[1]
    z = jnp.zeros((x.shape[0], s), jnp.float32)
    xs = jnp.concatenate([z, x[:, :w - s]], axis=1)
    col = lax.broadcasted_iota(jnp.int32, (1, w), 1)
    return xs * (col % SEQ >= s).astype(jnp.float32)


def _bn_apply(x, s1, s2, gam, bet):
    m = s1 * (1.0 / BL)
    v = s2 * (1.0 / BL) - m * m
    return jnp.maximum(gam * (x - m) * lax.rsqrt(v + EPS) + bet, 0.0)


CBB = CB // SEQ


def _shifted_cat(x, dil):
    return jnp.concatenate(
        [_shift_cols(x, 2 * dil), _shift_cols(x, dil), x], axis=0)


def _conv_body(xin, w_ref, b_ref, dil, o_ref, s1_ref, s2_ref):
    o = jnp.dot(w_ref[...], _shifted_cat(xin, dil),
                preferred_element_type=jnp.float32) + b_ref[...]
    o_ref[...] = o

    @pl.when(pl.program_id(0) == 0)
    def _():
        s1_ref[...] = jnp.zeros_like(s1_ref)
        s2_ref[...] = jnp.zeros_like(s2_ref)

    s1_ref[...] += jnp.sum(o, axis=1, keepdims=True)
    s2_ref[...] += jnp.sum(o * o, axis=1, keepdims=True)


def _col_spec(c):
    return pl.BlockSpec((c, CB), lambda j: (0, j))


def _full(a, b):
    return pl.BlockSpec((a, b), lambda j: (0, 0))


def _comb(o2_ref, s1_ref, s2_ref, g_ref, be_ref, x_ref, wd_ref, bd_ref):
    a2 = _bn_apply(o2_ref[...], s1_ref[...], s2_ref[...],
                   g_ref[...], be_ref[...])
    res = jnp.dot(wd_ref[...], x_ref[...],
                  preferred_element_type=jnp.float32) + bd_ref[...]
    return jnp.maximum(a2 + res, 0.0)


def _make_conv_a(cin, cout, dil):
    def _body(x_ref, w_ref, b_ref, o_ref, s1_ref, s2_ref):
        _conv_body(x_ref[...], w_ref, b_ref, dil, o_ref, s1_ref, s2_ref)

    def _call(x, w, b):
        return pl.pallas_call(
            _body, grid=(NCB,),
            in_specs=[_col_spec(cin), _full(cout, 3 * cin), _full(cout, 1)],
            out_specs=[_col_spec(cout), _full(cout, 1), _full(cout, 1)],
            out_shape=[jax.ShapeDtypeStruct((cout, BL), jnp.float32),
                       jax.ShapeDtypeStruct((cout, 1), jnp.float32),
                       jax.ShapeDtypeStruct((cout, 1), jnp.float32)],
        )(x, w, b)

    return _call


def _make_conv_b(cin, dil):
    def _body(x_ref, ps1_ref, ps2_ref, pg_ref, pb_ref, w_ref, b_ref,
              o_ref, s1_ref, s2_ref):
        xin = _bn_apply(x_ref[...], ps1_ref[...], ps2_ref[...],
                        pg_ref[...], pb_ref[...])
        _conv_body(xin, w_ref, b_ref, dil, o_ref, s1_ref, s2_ref)

    def _call(o1, s1, s2, pg, pb, w, b):
        return pl.pallas_call(
            _body, grid=(NCB,),
            in_specs=[_col_spec(cin)] + [_full(cin, 1)] * 4
            + [_full(cin, 3 * cin), _full(cin, 1)],
            out_specs=[_col_spec(cin), _full(cin, 1), _full(cin, 1)],
            out_shape=[jax.ShapeDtypeStruct((cin, BL), jnp.float32),
                       jax.ShapeDtypeStruct((cin, 1), jnp.float32),
                       jax.ShapeDtypeStruct((cin, 1), jnp.float32)],
        )(o1, s1, s2, pg, pb, w, b)

    return _call


def _make_conv_c(cres, cin, cout, dil):
    def _body(o2_ref, ps1_ref, ps2_ref, pg_ref, pb_ref, xr_ref, wd_ref,
              bd_ref, w_ref, b_ref, xin_ref, o_ref, s1_ref, s2_ref):
        xin = _comb(o2_ref, ps1_ref, ps2_ref, pg_ref, pb_ref, xr_ref,
                    wd_ref, bd_ref)
        xin_ref[...] = xin
        _conv_body(xin, w_ref, b_ref, dil, o_ref, s1_ref, s2_ref)

    def _call(o2, s1, s2, pg, pb, xr, wd, bd, w, b):
        return pl.pallas_call(
            _body, grid=(NCB,),
            in_specs=[_col_spec(cin)] + [_full(cin, 1)] * 4
            + [_col_spec(cres), _full(cin, cres), _full(cin, 1),
               _full(cout, 3 * cin), _full(cout, 1)],
            out_specs=[_col_spec(cin), _col_spec(cout), _full(cout, 1),
                       _full(cout, 1)],
            out_shape=[jax.ShapeDtypeStruct((cin, BL), jnp.float32),
                       jax.ShapeDtypeStruct((cout, BL), jnp.float32),
                       jax.ShapeDtypeStruct((cout, 1), jnp.float32),
                       jax.ShapeDtypeStruct((cout, 1), jnp.float32)],
        )(o2, s1, s2, pg, pb, xr, wd, bd, w, b)

    return _call


def _tail_body(o2_ref, ps1_ref, ps2_ref, pg_ref, pb_ref, xr_ref, wd_ref,
               bd_ref, w1_ref, b1_ref, w2_ref, b2_ref, out_ref):
    xin = _comb(o2_ref, ps1_ref, ps2_ref, pg_ref, pb_ref, xr_ref,
                wd_ref, bd_ref)
    ri = lax.broadcasted_iota(jnp.int32, (CB, CBB), 0)
    bi = lax.broadcasted_iota(jnp.int32, (CB, CBB), 1)
    sel = ((ri // SEQ == bi) & (ri % SEQ == SEQ - 1)).astype(jnp.float32)
    t = jnp.dot(xin, sel, preferred_element_type=jnp.float32)
    wc = jnp.dot(w1_ref[...], w2_ref[...],
                 preferred_element_type=jnp.float32)
    bc = jnp.dot(b1_ref[...], w2_ref[...],
                 preferred_element_type=jnp.float32) + b2_ref[...]
    out_ref[...] = lax.dot_general(
        t, wc, (((0,), (0,)), ((), ())),
        preferred_element_type=jnp.float32) + bc


def _tail(o2, s1, s2, pg, pb, xr, wd, bd, w1, b1, w2, b2):
    return pl.pallas_call(
        _tail_body, grid=(NCB,),
        in_specs=[_col_spec(32)] + [_full(32, 1)] * 4
        + [_col_spec(64), _full(32, 64), _full(32, 1),
           _full(32, 128), _full(1, 128), _full(128, 1), _full(1, 1)],
        out_specs=pl.BlockSpec((CBB, 1), lambda j: (j, 0)),
        out_shape=jax.ShapeDtypeStruct((BATCH, 1), jnp.float32),
    )(o2, s1, s2, pg, pb, xr, wd, bd, w1, b1, w2, b2)


_conv_a0 = _make_conv_a(NPG, 128, 1)
_conv_b0 = _make_conv_b(128, 1)
_conv_c1 = _make_conv_c(NPG, 128, 64, 2)
_conv_b1 = _make_conv_b(64, 2)
_conv_c2 = _make_conv_c(128, 64, 32, 4)
_conv_b2 = _make_conv_b(32, 4)


def _wcat(w3):
    return jnp.concatenate([w3[:, :, 0], w3[:, :, 1], w3[:, :, 2]], axis=1)


def _tcn_head(x0, p):
    o1, a1, a2 = _conv_a0(x0, _wcat(p['tcn0_w1']), p['tcn0_b1'][:, None])
    o2, b1, b2 = _conv_b0(o1, a1, a2, p['tcn0_bn1_g'][:, None],
                          p['tcn0_bn1_b'][:, None], _wcat(p['tcn0_w2']),
                          p['tcn0_b2'][:, None])
    x1, o3, c1, c2 = _conv_c1(o2, b1, b2, p['tcn0_bn2_g'][:, None],
                              p['tcn0_bn2_b'][:, None], x0,
                              p['tcn0_down_w'][:, :, 0],
                              p['tcn0_down_b'][:, None],
                              _wcat(p['tcn1_w1']), p['tcn1_b1'][:, None])
    o4, d1, d2 = _conv_b1(o3, c1, c2, p['tcn1_bn1_g'][:, None],
                          p['tcn1_bn1_b'][:, None], _wcat(p['tcn1_w2']),
                          p['tcn1_b2'][:, None])
    x2, o5, e1, e2 = _conv_c2(o4, d1, d2, p['tcn1_bn2_g'][:, None],
                              p['tcn1_bn2_b'][:, None], x1,
                              p['tcn1_down_w'][:, :, 0],
                              p['tcn1_down_b'][:, None],
                              _wcat(p['tcn2_w1']), p['tcn2_b1'][:, None])
    o6, f1, f2 = _conv_b2(o5, e1, e2, p['tcn2_bn1_g'][:, None],
                          p['tcn2_bn1_b'][:, None], _wcat(p['tcn2_w2']),
                          p['tcn2_b2'][:, None])
    ot = _tail(o6, f1, f2, p['tcn2_bn2_g'][:, None], p['tcn2_bn2_b'][:, None],
               x2, p['tcn2_down_w'][:, :, 0], p['tcn2_down_b'][:, None],
               p['fc1_w'], p['fc1_b'][None, :], p['fc_w'], p['fc_b'][None, :])
    return ot


def kernel(x, edge_index, params):
    p = params
    src = edge_index[0]
    dst = edge_index[1]
    dst2 = dst.reshape(N_EDGES // K_EDGE, K_EDGE)

    deg_part = _deg_counts(dst)
    deg = jnp.sum(deg_part[:, :N_NODES], axis=0) + 1.0
    dis = lax.rsqrt(deg)[:, None]

    pad64 = lambda a: jnp.pad(a, ((0, 0), (0, 64)))
    g1 = _t0(x, pad64(p['gcn1_w']), dis)
    pt = _msg128(src, dst2, g1)
    g2 = _tmid(pt[0], pt[1], g1, dis, pad64(p['gcn1_b'][None, :]),
               pad64(p['bn1_g'][None, :]), pad64(p['bn1_b'][None, :]),
               jnp.pad(p['gcn2_w'], ((0, 64), (0, 0))))
    pt = _msg128(src, dst2, g2)
    g3 = _tmid(pt[0], pt[1], g2, dis, p['gcn2_b'][None, :],
               p['bn2_g'][None, :], p['bn2_b'][None, :], p['gcn3_w'])
    pt = _msg128(src, dst2, g3)
    h3 = _tlast(pt[0], pt[1], g3, dis, p['gcn3_b'][None, :],
                p['bn3_g'][None, :], p['bn3_b'][None, :])

    x0 = h3.reshape(BATCH, NPG, SEQ).transpose(1, 0, 2).reshape(NPG, BL)
    return _tcn_head(x0, p)
